# baseline (speedup 1.0000x reference)
import functools

import jax
import jax.numpy as jnp
from jax.experimental import pallas as pl
from jax.experimental.pallas import tpu as pltpu
from jax.experimental.pallas import tpu_sc as plsc


def _tc_relayout(wt, v, d, nb):
    nsteps = pl.cdiv(v, nb)
    fold = 128 // d

    def body(x_ref, o_ref, scratch):
        scratch[...] = x_ref[...].T
        o_ref[...] = jnp.concatenate(
            [scratch[k::fold, :] for k in range(fold)], axis=1)

    return pl.pallas_call(
        body,
        grid=(nsteps,),
        in_specs=[pl.BlockSpec((d, nb), lambda i: (0, i))],
        out_specs=pl.BlockSpec((nb // fold, 128), lambda i: (i, 0)),
        out_shape=jax.ShapeDtypeStruct((v * d // 128, 128), jnp.float32),
        scratch_shapes=[pltpu.VMEM((nb, d), jnp.float32)],
        compiler_params=pltpu.CompilerParams(
            dimension_semantics=("parallel",)),
    )(wt)


def _sc_gather(table, idx2d, n, d):
    w = 128
    mesh = plsc.VectorSubcoreMesh(core_axis_name="c", subcore_axis_name="s")

    @functools.partial(
        pl.kernel,
        out_type=jax.ShapeDtypeStruct((n, d), jnp.float32),
        mesh=mesh,
        compiler_params=pltpu.CompilerParams(use_tc_tiling_on_sc=False),
    )
    def gather_kernel(table_hbm, idx_hbm, out_hbm):
        def body(idx_vmem, out_vmem):
            pltpu.sync_copy(table_hbm.at[idx_vmem.at[0]], out_vmem)

        pltpu.emit_pipeline(
            body,
            grid=(n // w,),
            in_specs=[pl.BlockSpec((1, w), lambda i: (0, i))],
            out_specs=[pl.BlockSpec((w, d), lambda i: (i, 0))],
            core_axis_name=("c", "s"),
            dimension_semantics=(pltpu.PARALLEL,),
        )(idx_hbm, out_hbm)

    return gather_kernel(table, idx2d)


def _tc_loss(g_lin, a_mat, t_mat, s, dim, b, bb):
    sd = s * dim
    fold = sd // 128
    nsteps = b // bb
    dn = (((1,), (0,)), ((), ()))

    def body(g_ref, a_ref, t_ref, out_ref):
        i = pl.program_id(0)
        g = jnp.concatenate(
            [g_ref[k::fold, :] for k in range(fold)], axis=1)
        a = a_ref[...]
        t = t_ref[...]
        u = g[:, 0:dim]
        util = jax.lax.dot_general(u, t, dn, preferred_element_type=jnp.float32)
        diff = g - util
        sq_all = jax.lax.dot_general(
            diff * diff, a, dn, preferred_element_type=jnp.float32)
        ns_all = jax.lax.dot_general(
            g * g, a, dn, preferred_element_type=jnp.float32)
        un = ns_all[:, 0:1]
        vn = ns_all[:, 1:s]
        sq = sq_all[:, 1:s]
        eps = 1e-7
        denom = jnp.maximum((1.0 - un) * (1.0 - vn), eps)
        arg = jnp.maximum(1.0 + 2.0 * sq / denom, 1.0 + eps)
        dist = -jnp.log(arg + jnp.sqrt((arg - 1.0) * (arg + 1.0)))
        m = jnp.max(dist, axis=1, keepdims=True)
        lz = jnp.log(jnp.sum(jnp.exp(dist - m), axis=1, keepdims=True)) + m
        part = jnp.sum(lz - dist[:, 0:1])

        @pl.when(i == 0)
        def _():
            out_ref[...] = jnp.zeros_like(out_ref)

        out_ref[...] = out_ref[...] + part

    return pl.pallas_call(
        body,
        grid=(nsteps,),
        in_specs=[
            pl.BlockSpec((fold * bb, 128), lambda i: (i, 0)),
            pl.BlockSpec((sd, s), lambda i: (0, 0)),
            pl.BlockSpec((dim, sd), lambda i: (0, 0)),
        ],
        out_specs=pl.BlockSpec((1, 1), lambda i: (0, 0)),
        out_shape=jax.ShapeDtypeStruct((1, 1), jnp.float32),
    )(g_lin, a_mat, t_mat)


def kernel(pred, succ, neg, weight):
    b = pred.shape[0]
    nneg = neg.shape[1]
    dim = weight.shape[1]
    s = 2 + nneg
    sd = s * dim

    idx = jnp.concatenate(
        [succ[:, None], pred[:, None], neg], axis=1
    ).astype(jnp.int32)
    n = s * b
    v = weight.shape[0]
    w_row = _tc_relayout(weight.T, v, dim, nb=8192).reshape(v, dim)
    idx_flat = idx.reshape(n)
    nsplit = 8
    half = n // nsplit
    gathered = [
        _sc_gather(w_row, idx_flat[h * half:(h + 1) * half].reshape(1, half),
                   half, dim)
        for h in range(nsplit)
    ]

    cols = jax.lax.iota(jnp.int32, sd)
    a_mat = (cols[:, None] // dim == jax.lax.iota(jnp.int32, s)[None, :])
    a_mat = a_mat.astype(jnp.float32)
    t_mat = (jax.lax.iota(jnp.int32, dim)[:, None] == cols[None, :] % dim)
    t_mat = t_mat.astype(jnp.float32)

    hb = b // nsplit
    parts = []
    for h in range(nsplit):
        gh = gathered[h]
        g_lin = gh.reshape(hb * sd // 128, 128)
        parts.append(_tc_loss(g_lin, a_mat, t_mat, s, dim, hb, bb=1024))
    loss = sum(parts) * (1.0 / b)
    return loss.reshape(())


# speedup vs baseline: 1.0214x; 1.0214x over previous
import functools

import jax
import jax.numpy as jnp
from jax.experimental import pallas as pl
from jax.experimental.pallas import tpu as pltpu
from jax.experimental.pallas import tpu_sc as plsc


def _tc_relayout(wt, v, d, nb):
    nsteps = pl.cdiv(v, nb)
    fold = 128 // d

    def body(x_ref, o_ref, scratch):
        scratch[...] = x_ref[...].T
        o_ref[...] = jnp.concatenate(
            [scratch[k::fold, :] for k in range(fold)], axis=1)

    return pl.pallas_call(
        body,
        grid=(nsteps,),
        in_specs=[pl.BlockSpec((d, nb), lambda i: (0, i))],
        out_specs=pl.BlockSpec((nb // fold, 128), lambda i: (i, 0)),
        out_shape=jax.ShapeDtypeStruct((v * d // 128, 128), jnp.float32),
        scratch_shapes=[pltpu.VMEM((nb, d), jnp.float32)],
        compiler_params=pltpu.CompilerParams(
            dimension_semantics=("parallel",)),
    )(wt)


def _sc_gather(table, idx2d, n, d):
    w = 128
    mesh = plsc.VectorSubcoreMesh(core_axis_name="c", subcore_axis_name="s")

    @functools.partial(
        pl.kernel,
        out_type=jax.ShapeDtypeStruct((n, d), jnp.float32),
        mesh=mesh,
        compiler_params=pltpu.CompilerParams(use_tc_tiling_on_sc=False),
    )
    def gather_kernel(table_hbm, idx_hbm, out_hbm):
        def body(idx_vmem, out_vmem):
            pltpu.sync_copy(table_hbm.at[idx_vmem.at[0]], out_vmem)

        pltpu.emit_pipeline(
            body,
            grid=(n // w,),
            in_specs=[pl.BlockSpec((1, w), lambda i: (0, i))],
            out_specs=[pl.BlockSpec((w, d), lambda i: (i, 0))],
            core_axis_name=("c", "s"),
            dimension_semantics=(pltpu.PARALLEL,),
        )(idx_hbm, out_hbm)

    return gather_kernel(table, idx2d)


def _tc_loss(g_lin, a_mat, t_mat, s, dim, b, bb):
    sd = s * dim
    fold = sd // 128
    nsteps = b // bb
    dn = (((1,), (0,)), ((), ()))

    def body(g_ref, a_ref, t_ref, out_ref):
        i = pl.program_id(0)
        g = jnp.concatenate(
            [g_ref[k::fold, :] for k in range(fold)], axis=1)
        a = a_ref[...]
        t = t_ref[...]
        u = g[:, 0:dim]
        util = jax.lax.dot_general(u, t, dn, preferred_element_type=jnp.float32)
        diff = g - util
        sq_all = jax.lax.dot_general(
            diff * diff, a, dn, preferred_element_type=jnp.float32)
        ns_all = jax.lax.dot_general(
            g * g, a, dn, preferred_element_type=jnp.float32)
        un = ns_all[:, 0:1]
        vn = ns_all[:, 1:s]
        sq = sq_all[:, 1:s]
        eps = 1e-7
        denom = jnp.maximum((1.0 - un) * (1.0 - vn), eps)
        arg = jnp.maximum(1.0 + 2.0 * sq / denom, 1.0 + eps)
        dist = -jnp.log(arg + jnp.sqrt((arg - 1.0) * (arg + 1.0)))
        m = jnp.max(dist, axis=1, keepdims=True)
        lz = jnp.log(jnp.sum(jnp.exp(dist - m), axis=1, keepdims=True)) + m
        part = jnp.sum(lz - dist[:, 0:1])

        @pl.when(i == 0)
        def _():
            out_ref[...] = jnp.zeros_like(out_ref)

        out_ref[...] = out_ref[...] + part

    return pl.pallas_call(
        body,
        grid=(nsteps,),
        in_specs=[
            pl.BlockSpec((fold * bb, 128), lambda i: (i, 0)),
            pl.BlockSpec((sd, s), lambda i: (0, 0)),
            pl.BlockSpec((dim, sd), lambda i: (0, 0)),
        ],
        out_specs=pl.BlockSpec((1, 1), lambda i: (0, 0)),
        out_shape=jax.ShapeDtypeStruct((1, 1), jnp.float32),
    )(g_lin, a_mat, t_mat)


def kernel(pred, succ, neg, weight):
    b = pred.shape[0]
    nneg = neg.shape[1]
    dim = weight.shape[1]
    s = 2 + nneg
    sd = s * dim

    idx = jnp.concatenate(
        [succ[:, None], pred[:, None], neg], axis=1
    ).astype(jnp.int32)
    n = s * b
    v = weight.shape[0]
    w_row = _tc_relayout(weight.T, v, dim, nb=8192).reshape(v, dim)
    idx_flat = idx.reshape(n)
    nsplit = 4
    half = n // nsplit
    gathered = [
        _sc_gather(w_row, idx_flat[h * half:(h + 1) * half].reshape(1, half),
                   half, dim)
        for h in range(nsplit)
    ]

    cols = jax.lax.iota(jnp.int32, sd)
    a_mat = (cols[:, None] // dim == jax.lax.iota(jnp.int32, s)[None, :])
    a_mat = a_mat.astype(jnp.float32)
    t_mat = (jax.lax.iota(jnp.int32, dim)[:, None] == cols[None, :] % dim)
    t_mat = t_mat.astype(jnp.float32)

    hb = b // nsplit
    parts = []
    for h in range(nsplit):
        gh = gathered[h]
        g_lin = gh.reshape(hb * sd // 128, 128)
        parts.append(_tc_loss(g_lin, a_mat, t_mat, s, dim, hb, bb=1024))
    loss = sum(parts) * (1.0 / b)
    return loss.reshape(())
